# Initial kernel scaffold; baseline (speedup 1.0000x reference)
import jax
import jax.numpy as jnp
from jax.experimental import pallas as pl


def kernel(inputs):
    raise NotImplementedError("write your pallas kernel here")


# speedup vs baseline: 1.2331x; 1.2331x over previous
import functools

import jax
import jax.numpy as jnp
from jax import lax
from jax.experimental import pallas as pl
from jax.experimental.pallas import tpu as pltpu
from jax.experimental.pallas import tpu_sc as plsc

DST_NUM = 16
N_TOK = 8192
D_MODEL = 2048
CAPACITY = 2 * N_TOK // DST_NUM
N_SLOTS = DST_NUM * CAPACITY

NW = 32
TPW = N_TOK // NW
CHUNK = 16
NCHUNK = TPW // CHUNK


def _routing():
    skey = jax.random.key(42)
    score = jax.random.normal(skey, (N_TOK, DST_NUM), dtype=jnp.float32)
    _, top_idx = jax.lax.top_k(score, 1)
    dst = top_idx[:, 0]
    onehot = (dst[:, None] == jnp.arange(DST_NUM)[None, :]).astype(jnp.int32)
    pos = jnp.cumsum(onehot, axis=0) - 1
    pos_in_expert = jnp.take_along_axis(pos, dst[:, None], axis=1)[:, 0]
    flat_dst = (dst * CAPACITY + pos_in_expert).astype(jnp.int32)
    filled = jnp.zeros((N_SLOTS,), jnp.bool_).at[flat_dst].set(True)
    zero_slots = jnp.nonzero(~filled, size=N_SLOTS - N_TOK, fill_value=0)[0]
    return flat_dst, zero_slots.astype(jnp.int32)


@functools.partial(
    pl.kernel,
    out_type=jax.ShapeDtypeStruct((N_SLOTS, D_MODEL), jnp.float32),
    mesh=plsc.VectorSubcoreMesh(core_axis_name="c", subcore_axis_name="s"),
    scratch_types=[
        pltpu.VMEM((NCHUNK, CHUNK), jnp.int32),
        pltpu.VMEM((NCHUNK, CHUNK), jnp.int32),
        pltpu.VMEM((CHUNK, D_MODEL), jnp.float32),
        pltpu.VMEM((CHUNK, D_MODEL), jnp.float32),
        pltpu.SemaphoreType.DMA,
    ],
)
def _scatter_kernel(in_hbm, fd_hbm, zd_hbm, zsrc_hbm, out_hbm,
                    idx_v, zidx_v, buf, zbuf, sem):
    wid = lax.axis_index("s") * 2 + lax.axis_index("c")
    pltpu.sync_copy(fd_hbm.at[pl.ds(wid * NCHUNK, NCHUNK)], idx_v)
    pltpu.sync_copy(zd_hbm.at[pl.ds(wid * NCHUNK, NCHUNK)], zidx_v)
    pltpu.sync_copy(zsrc_hbm, zbuf)

    base = wid * TPW
    for j in range(NCHUNK):
        pltpu.sync_copy(in_hbm.at[pl.ds(base + j * CHUNK, CHUNK)], buf)
        pltpu.async_copy(buf, out_hbm.at[idx_v.at[j]], sem).wait()
    for j in range(NCHUNK):
        pltpu.async_copy(zbuf, out_hbm.at[zidx_v.at[j]], sem).wait()


def kernel(inputs):
    flat_dst, zero_slots = _routing()
    fd = flat_dst.reshape(NW * NCHUNK, CHUNK)
    zd = zero_slots.reshape(NW * NCHUNK, CHUNK)
    zsrc = jnp.zeros((CHUNK, D_MODEL), jnp.float32)
    out = _scatter_kernel(inputs, fd, zd, zsrc)
    return out.reshape(DST_NUM, CAPACITY, D_MODEL)
